# baseline (speedup 1.0000x reference)
import functools

import jax
import jax.numpy as jnp
from jax import lax
from jax.experimental import pallas as pl
from jax.experimental.pallas import tpu as pltpu
from jax.experimental.pallas import tpu_sc as plsc

NC = 2
NS = 16
NW = NC * NS
CHUNK = 128
H = 32
G = 64


def _proj(x, w, b, bn):
    n, cin = x.shape
    def body(xr, wr, br, outr):
        o = jnp.dot(xr[...], wr[...], preferred_element_type=jnp.float32)
        outr[...] = jnp.maximum(o + br[...], 0.0)
    return pl.pallas_call(
        body,
        grid=(n // bn,),
        in_specs=[pl.BlockSpec((bn, cin), lambda i: (i, 0)),
                  pl.BlockSpec((cin, H), lambda i: (0, 0)),
                  pl.BlockSpec((1, H), lambda i: (0, 0))],
        out_specs=pl.BlockSpec((bn, H), lambda i: (i, 0)),
        out_shape=jax.ShapeDtypeStruct((n, H), jnp.float32),
    )(x, w, b.reshape(1, H))


def _msg(ea, hs, w1, b1, w2b, b2row, be):
    ep, edp = ea.shape
    def body(ear, hsr, w1r, b1r, w2r, b2r, outr):
        a = jnp.dot(ear[...], w1r[...], preferred_element_type=jnp.float32)
        a = jnp.maximum(a + b1r[...], 0.0).astype(jnp.bfloat16)
        we = jnp.dot(a, w2r[...], preferred_element_type=jnp.float32)
        we = we + b2r[...]
        hsv = hsr[...]
        acc = hsv[:, 0:1] * we[:, 0:H]
        for i in range(1, H):
            acc = acc + hsv[:, i:i + 1] * we[:, i * H:(i + 1) * H]
        outr[...] = acc
    return pl.pallas_call(
        body,
        grid=(ep // be,),
        in_specs=[pl.BlockSpec((be, edp), lambda i: (i, 0)),
                  pl.BlockSpec((be, H), lambda i: (i, 0)),
                  pl.BlockSpec((edp, H), lambda i: (0, 0)),
                  pl.BlockSpec((1, H), lambda i: (0, 0)),
                  pl.BlockSpec((H, H * H), lambda i: (0, 0)),
                  pl.BlockSpec((1, H * H), lambda i: (0, 0))],
        out_specs=pl.BlockSpec((be, H), lambda i: (i, 0)),
        out_shape=jax.ShapeDtypeStruct((ep, H), jnp.float32),
    )(ea, hs, w1, b1.reshape(1, H), w2b, b2row)


def _sc_gather(h, srcm, ep):
    cw = ep // (NW * CHUNK)
    mesh = plsc.VectorSubcoreMesh(core_axis_name="c", subcore_axis_name="s", num_cores=NC, num_subcores=NS)

    @functools.partial(
        pl.kernel,
        out_type=jax.ShapeDtypeStruct((ep, H), jnp.float32),
        mesh=mesh,
        scratch_types=[pltpu.VMEM((cw, CHUNK), jnp.int32),
                       pltpu.VMEM((CHUNK, H), jnp.float32),
                       pltpu.SemaphoreType.DMA,
                       pltpu.SemaphoreType.DMA],
        compiler_params=pltpu.CompilerParams(use_tc_tiling_on_sc=False),
    )
    def k(h_hbm, srcm_hbm, hs_hbm, idx_v, rows_v, gsem, ssem):
        wid = lax.axis_index("s") * NC + lax.axis_index("c")
        c0 = wid * cw
        pltpu.sync_copy(srcm_hbm.at[pl.ds(c0, cw)], idx_v)

        def step(t, carry):
            pltpu.async_copy(h_hbm.at[idx_v.at[t]], rows_v, gsem).wait()
            pltpu.async_copy(
                rows_v, hs_hbm.at[pl.ds((c0 + t) * CHUNK, CHUNK)], ssem
            ).wait()
            return carry
        lax.fori_loop(0, cw, step, 0)

    return k(h, srcm)


def _sc_scatter(msg, dstm, z_nph, npad, with_cnt, z_np16=None, ones_c16=None):
    ep = msg.shape[0]
    cw = ep // (NW * CHUNK)
    rps = npad // NS
    mesh = plsc.VectorSubcoreMesh(core_axis_name="c", subcore_axis_name="s", num_cores=NC, num_subcores=NS)

    out_type = [jax.ShapeDtypeStruct((NC, npad, H), jnp.float32)]
    scratch = [pltpu.VMEM((cw, CHUNK), jnp.int32),
               pltpu.VMEM((CHUNK, H), jnp.float32),
               pltpu.VMEM_SHARED((npad, H), jnp.float32),
               pltpu.SemaphoreType.DMA]
    if with_cnt:
        out_type.append(jax.ShapeDtypeStruct((NC, npad, 16), jnp.float32))
        scratch += [pltpu.VMEM((CHUNK, 16), jnp.float32),
                    pltpu.VMEM_SHARED((npad, 16), jnp.float32)]

    def body(refs):
        if with_cnt:
            (msg_hbm, dstm_hbm, z_hbm, zc_hbm, ones_hbm,
             acc_hbm, cnt_hbm,
             idx_v, msg_v, acc_sh, sem, ones_v, cnt_sh) = refs
        else:
            (msg_hbm, dstm_hbm, z_hbm,
             acc_hbm,
             idx_v, msg_v, acc_sh, sem) = refs
        cid = lax.axis_index("c")
        sid = lax.axis_index("s")
        wid = sid * NC + cid
        row0 = sid * rps
        pltpu.sync_copy(z_hbm.at[pl.ds(row0, rps)], acc_sh.at[pl.ds(row0, rps)])
        if with_cnt:
            pltpu.sync_copy(zc_hbm.at[pl.ds(row0, rps)],
                            cnt_sh.at[pl.ds(row0, rps)])
            pltpu.sync_copy(ones_hbm, ones_v)
        plsc.subcore_barrier()
        c0 = wid * cw
        pltpu.sync_copy(dstm_hbm.at[pl.ds(c0, cw)], idx_v)

        def step(t, carry):
            pltpu.sync_copy(msg_hbm.at[pl.ds((c0 + t) * CHUNK, CHUNK)], msg_v)
            pltpu.sync_copy(msg_v, acc_sh.at[idx_v.at[t]], add=True)
            if with_cnt:
                pltpu.sync_copy(ones_v, cnt_sh.at[idx_v.at[t]], add=True)
            return carry
        lax.fori_loop(0, cw, step, 0)
        plsc.subcore_barrier()
        pltpu.sync_copy(acc_sh.at[pl.ds(row0, rps)],
                        acc_hbm.at[cid].at[pl.ds(row0, rps)])
        if with_cnt:
            pltpu.sync_copy(cnt_sh.at[pl.ds(row0, rps)],
                            cnt_hbm.at[cid].at[pl.ds(row0, rps)])

    if with_cnt:
        def k6(msg_hbm, dstm_hbm, z_hbm, zc_hbm, ones_hbm, acc_hbm, cnt_hbm,
               idx_v, msg_v, acc_sh, sem, ones_v, cnt_sh):
            body((msg_hbm, dstm_hbm, z_hbm, zc_hbm, ones_hbm, acc_hbm,
                  cnt_hbm, idx_v, msg_v, acc_sh, sem, ones_v, cnt_sh))
        kf = pl.kernel(k6, out_type=tuple(out_type), mesh=mesh,
                       scratch_types=scratch,
                       compiler_params=pltpu.CompilerParams(
                           use_tc_tiling_on_sc=False))
        return kf(msg, dstm, z_nph, z_np16, ones_c16)
    else:
        def k4(msg_hbm, dstm_hbm, z_hbm, acc_hbm,
               idx_v, msg_v, acc_sh, sem):
            body((msg_hbm, dstm_hbm, z_hbm, acc_hbm,
                  idx_v, msg_v, acc_sh, sem))
        kf = pl.kernel(k4, out_type=out_type[0], mesh=mesh,
                       scratch_types=scratch,
                       compiler_params=pltpu.CompilerParams(
                           use_tc_tiling_on_sc=False))
        return kf(msg, dstm, z_nph)


def _update(acc, cnt, h, root, bias, gamma_s, beta, bn):
    n = h.shape[0]
    def body(accr, cntr, hr, rootr, biasr, gammar, betar, outr):
        s = accr[0] + accr[1]
        c = cntr[0, :, 0:1] + cntr[1, :, 0:1]
        agg = s / jnp.clip(c, 1.0, None)
        o = agg + jnp.dot(hr[...], rootr[...],
                          preferred_element_type=jnp.float32) + biasr[...]
        outr[...] = jnp.maximum(o * gammar[...] + betar[...], 0.0)
    return pl.pallas_call(
        body,
        grid=(n // bn,),
        in_specs=[pl.BlockSpec((NC, bn, H), lambda i: (0, i, 0)),
                  pl.BlockSpec((NC, bn, 16), lambda i: (0, i, 0)),
                  pl.BlockSpec((bn, H), lambda i: (i, 0)),
                  pl.BlockSpec((H, H), lambda i: (0, 0)),
                  pl.BlockSpec((1, H), lambda i: (0, 0)),
                  pl.BlockSpec((1, H), lambda i: (0, 0)),
                  pl.BlockSpec((1, H), lambda i: (0, 0))],
        out_specs=pl.BlockSpec((bn, H), lambda i: (i, 0)),
        out_shape=jax.ShapeDtypeStruct((n, H), jnp.float32),
    )(acc, cnt, h, root, bias.reshape(1, H), gamma_s.reshape(1, H),
      beta.reshape(1, H))


def _pool_cls(h, batch2d, w1, b1, w2row, b2):
    n = h.shape[0]
    nchunks, bn = batch2d.shape
    def body(hr, br, w1r, b1r, w2r, b2r, outr):
        pooled = jnp.zeros((G, H), jnp.float32)
        gcnt = jnp.zeros((G, 1), jnp.float32)
        for c in range(nchunks):
            bv = br[pl.ds(c, 1), :]
            oh = (lax.broadcasted_iota(jnp.int32, (G, bn), 0) == bv)
            ohf = oh.astype(jnp.float32)
            pooled = pooled + jnp.dot(ohf, hr[pl.ds(c * bn, bn), :],
                                      preferred_element_type=jnp.float32)
            gcnt = gcnt + jnp.sum(ohf, axis=1, keepdims=True)
        pooled = pooled / jnp.clip(gcnt, 1.0, None)
        z = jnp.dot(pooled, w1r[...], preferred_element_type=jnp.float32)
        z = jnp.maximum(z + b1r[...], 0.0)
        zf = jnp.sum(z * w2r[...], axis=1, keepdims=True) + b2r[...]
        outr[...] = zf
    return pl.pallas_call(
        body,
        in_specs=[pl.BlockSpec((n, H), lambda: (0, 0)),
                  pl.BlockSpec((nchunks, bn), lambda: (0, 0)),
                  pl.BlockSpec((H, G), lambda: (0, 0)),
                  pl.BlockSpec((1, G), lambda: (0, 0)),
                  pl.BlockSpec((1, G), lambda: (0, 0)),
                  pl.BlockSpec((1, 1), lambda: (0, 0))],
        out_specs=pl.BlockSpec((G, 1), lambda: (0, 0)),
        out_shape=jax.ShapeDtypeStruct((G, 1), jnp.float32),
    )(h, batch2d, w1, b1.reshape(1, G), w2row, b2.reshape(1, 1))


def kernel(x, edge_index, batch, edge_attr, params):
    n, _ = x.shape
    e = edge_index.shape[1]
    ed = edge_attr.shape[1]
    edp = 8
    epw = NW * CHUNK
    ep = ((e + epw - 1) // epw) * epw
    npad = ((n + 1 + NS - 1) // NS) * NS
    pad = ep - e

    src = jnp.concatenate([edge_index[0], jnp.zeros((pad,), jnp.int32)])
    dstv = jnp.concatenate([edge_index[1], jnp.full((pad,), n, jnp.int32)])
    srcm = src.reshape(ep // CHUNK, CHUNK)
    dstm = dstv.reshape(ep // CHUNK, CHUNK)
    ea = jnp.zeros((ep, edp), jnp.float32).at[:e, :ed].set(edge_attr)
    z_nph = jnp.zeros((npad, H), jnp.float32)
    z_np16 = jnp.zeros((npad, 16), jnp.float32)
    ones_c16 = jnp.ones((CHUNK, 16), jnp.float32)
    inv_std = 1.0 / jnp.sqrt(1.0 + 1e-5)

    p = params
    h = _proj(x, p['lin_in_w'], p['lin_in_b'], 2000)
    cnt = None
    for li, lp in enumerate(p['layers']):
        w1p = jnp.zeros((edp, H), jnp.float32).at[:ed].set(lp['enn_w1'])
        w2b = lp['enn_w2'].astype(jnp.bfloat16)
        b2row = lp['enn_b2'].reshape(1, H * H)
        hs = _sc_gather(h, srcm, ep)
        msg = _msg(ea, hs, w1p, lp['enn_b1'], w2b, b2row, 2048)
        if li == 0:
            acc, cnt = _sc_scatter(msg, dstm, z_nph, npad, True,
                                   z_np16, ones_c16)
        else:
            acc = _sc_scatter(msg, dstm, z_nph, npad, False)
        h = _update(acc, cnt, h, lp['root'], lp['bias'],
                    lp['gamma'] * inv_std, lp['beta'], 2000)
    b2d = batch.reshape(8, n // 8)
    out = _pool_cls(h, b2d, p['cls_w1'], p['cls_b1'],
                    p['cls_w2'].reshape(1, G), p['cls_b2'])
    return out[:, 0]
